# baseline (speedup 1.0000x reference)
import functools
import math

import jax
import jax.numpy as jnp
import numpy as np
from jax import lax
from jax.experimental import pallas as pl
from jax.experimental.pallas import tpu as pltpu
from jax.experimental.pallas import tpu_sc as plsc

_MAX_LEN = 200


def _pe_table(max_len, d_model):
    position = np.arange(0, max_len, dtype=np.float32)[:, None]
    div_term = np.exp(
        np.arange(0, d_model, 2, dtype=np.float32) * (-math.log(10000.0) / d_model)
    )
    pe = np.zeros((max_len, d_model), dtype=np.float32)
    pe[:, 0::2] = np.sin(position * div_term)
    if d_model % 2 == 1:
        pe[:, 1::2] = np.cos(position * div_term[:-1])
    else:
        pe[:, 1::2] = np.cos(position * div_term)
    return pe


@functools.partial(jax.jit, static_argnames=("batch", "seq", "d"))
def _embed_pe(table, x, pe, *, batch, seq, d):
    NC, NS = 2, 16
    NW = NC * NS
    assert batch == NW * 128, batch
    assert d % 8 == 0, d
    CR = d // 8
    assert seq % 4 == 0, seq

    mesh = plsc.VectorSubcoreMesh(core_axis_name="c", subcore_axis_name="s")

    @functools.partial(
        pl.kernel,
        mesh=mesh,
        out_type=jax.ShapeDtypeStruct((seq, CR, NW, 8, 128), jnp.float32),
        compiler_params=pltpu.CompilerParams(
            use_tc_tiling_on_sc=False, needs_layout_passes=False
        ),
        scratch_types=[
            pltpu.VMEM((128, seq), jnp.int32),
            pltpu.VMEM((seq, 128), jnp.int32),
            pltpu.VMEM((256, d), jnp.float32),
            pltpu.VMEM((256, d), jnp.float32),
            pltpu.VMEM((2, CR, 8, 128), jnp.float32),
            pltpu.VMEM((2, CR, 8, 128), jnp.float32),
            pltpu.VMEM((seq, d), jnp.float32),
            pltpu.SemaphoreType.DMA,
            pltpu.SemaphoreType.DMA,
        ],
    )
    def k(table_hbm, x_hbm, pe_hbm, out_hbm,
          xin, idxT, g0, g1, t0, t1, pe_v, gsem, ssem):
        g_b = (g0, g1)
        t_b = (t0, t1)
        wid = lax.axis_index("s") * NC + lax.axis_index("c")
        wb = wid * 128

        pltpu.sync_copy(pe_hbm, pe_v)
        pltpu.sync_copy(x_hbm.at[pl.ds(wb, 128)], xin)

        iota = lax.iota(jnp.int32, 16)

        def tr_body(l, _):
            cols = jnp.full((16,), l, jnp.int32)
            for bb in range(8):
                v = plsc.load_gather(xin, [bb * 16 + iota, cols])
                idxT[l, pl.ds(bb * 16, 16)] = v
            return 0

        lax.fori_loop(0, seq, tr_body, 0)

        def fire(j, p):
            pltpu.async_copy(
                table_hbm.at[idxT.at[2 * j]], g_b[p].at[pl.ds(0, 128)], gsem
            )
            pltpu.async_copy(
                table_hbm.at[idxT.at[2 * j + 1]], g_b[p].at[pl.ds(128, 128)], gsem
            )

        def drain(j, p):
            pltpu.make_async_copy(
                table_hbm.at[idxT.at[2 * j]], g_b[p].at[pl.ds(0, 128)], gsem
            ).wait()
            pltpu.make_async_copy(
                table_hbm.at[idxT.at[2 * j + 1]], g_b[p].at[pl.ds(128, 128)], gsem
            ).wait()

        def store(j, p):
            pltpu.async_copy(
                t_b[p], out_hbm.at[pl.ds(2 * j, 2)].at[:, :, wid], ssem
            )

        def wait_store(j, p):
            pltpu.make_async_copy(
                t_b[p], out_hbm.at[pl.ds(2 * j, 2)].at[:, :, wid], ssem
            ).wait()

        def compute(j, p):
            for h in range(2):
                rows_l = jnp.full((16,), 2 * j + h, jnp.int32)
                for cr in range(CR):
                    for ci in range(8):
                        c = cr * 8 + ci
                        cols = jnp.full((16,), c, jnp.int32)
                        pec = plsc.load_gather(pe_v, [rows_l, cols])
                        for bb in range(8):
                            v = plsc.load_gather(
                                g_b[p], [h * 128 + bb * 16 + iota, cols]
                            )
                            t_b[p][h, cr, ci, pl.ds(bb * 16, 16)] = v + pec

        nj = seq // 2
        fire(0, 0)

        def step2(i, _):
            j0 = i * 2
            for p in (0, 1):
                j = j0 + p

                @pl.when(j + 1 < nj)
                def _():
                    fire(j + 1, 1 - p)

                @pl.when(j >= 2)
                def _():
                    wait_store(j - 2, p)

                drain(j, p)
                compute(j, p)
                store(j, p)
            return 0

        lax.fori_loop(0, nj // 2, step2, 0)
        wait_store(nj - 2, 0)
        wait_store(nj - 1, 1)

    return k(table, x, pe)


def kernel(x, table):
    batch, seq = x.shape
    _, d = table.shape
    pe = jnp.asarray(_pe_table(_MAX_LEN, d)[:seq])
    out5 = _embed_pe(table, x, pe, batch=batch, seq=seq, d=d)
    return jnp.transpose(out5, (2, 4, 0, 1, 3)).reshape(batch, seq, d)


# speedup vs baseline: 1.2002x; 1.2002x over previous
import functools
import math

import jax
import jax.numpy as jnp
import numpy as np
from jax import lax
from jax.experimental import pallas as pl
from jax.experimental.pallas import tpu as pltpu
from jax.experimental.pallas import tpu_sc as plsc

_MAX_LEN = 200


def _pe_table(max_len, d_model):
    position = np.arange(0, max_len, dtype=np.float32)[:, None]
    div_term = np.exp(
        np.arange(0, d_model, 2, dtype=np.float32) * (-math.log(10000.0) / d_model)
    )
    pe = np.zeros((max_len, d_model), dtype=np.float32)
    pe[:, 0::2] = np.sin(position * div_term)
    if d_model % 2 == 1:
        pe[:, 1::2] = np.cos(position * div_term[:-1])
    else:
        pe[:, 1::2] = np.cos(position * div_term)
    return pe


@functools.partial(jax.jit, static_argnames=("batch", "seq", "d"))
def _embed_pe(table, x, pe, *, batch, seq, d):
    NC, NS = 2, 16
    NW = NC * NS
    assert batch == NW * 128, batch
    assert d % 16 == 0, d
    CR = d // 8
    DH = d // 16
    assert seq % 8 == 0, seq

    mesh = plsc.VectorSubcoreMesh(core_axis_name="c", subcore_axis_name="s")

    @functools.partial(
        pl.kernel,
        mesh=mesh,
        out_type=jax.ShapeDtypeStruct((seq, CR, NW, 8, 128), jnp.float32),
        compiler_params=pltpu.CompilerParams(
            use_tc_tiling_on_sc=False, needs_layout_passes=False
        ),
        scratch_types=[
            pltpu.VMEM((128, seq), jnp.int32),
            pltpu.VMEM((seq, 128), jnp.int32),
            pltpu.VMEM((256, d), jnp.float32),
            pltpu.VMEM((256, d), jnp.float32),
            pltpu.VMEM((2, CR, 8, 128), jnp.float32),
            pltpu.VMEM((2, CR, 8, 128), jnp.float32),
            pltpu.VMEM((seq, d), jnp.float32),
            pltpu.SemaphoreType.DMA,
            pltpu.SemaphoreType.DMA,
        ],
    )
    def k(table_hbm, x_hbm, pe_hbm, out_hbm,
          xin, idxT, g0, g1, t0, t1, pe_v, gsem, ssem):
        g_b = (g0, g1)
        t_b = (t0, t1)
        wid = lax.axis_index("s") * NC + lax.axis_index("c")
        wb = wid * 128

        pltpu.sync_copy(pe_hbm, pe_v)
        pltpu.sync_copy(x_hbm.at[pl.ds(wb, 128)], xin)

        iota = lax.iota(jnp.int32, 16)

        def tr_body(l, _):
            cols = jnp.full((16,), l, jnp.int32)
            for bb in range(8):
                v = plsc.load_gather(xin, [bb * 16 + iota, cols])
                idxT[l, pl.ds(bb * 16, 16)] = v
            return 0

        lax.fori_loop(0, seq, tr_body, 0)

        def fire(j, p):
            pltpu.async_copy(
                table_hbm.at[idxT.at[2 * j]], g_b[p].at[pl.ds(0, 128)], gsem
            )
            pltpu.async_copy(
                table_hbm.at[idxT.at[2 * j + 1]], g_b[p].at[pl.ds(128, 128)], gsem
            )

        def drain(j, p):
            pltpu.make_async_copy(
                table_hbm.at[idxT.at[2 * j]], g_b[p].at[pl.ds(0, 128)], gsem
            ).wait()
            pltpu.make_async_copy(
                table_hbm.at[idxT.at[2 * j + 1]], g_b[p].at[pl.ds(128, 128)], gsem
            ).wait()

        def store(j, p):
            pltpu.async_copy(
                t_b[p], out_hbm.at[pl.ds(2 * j, 2)].at[:, :, wid], ssem
            )

        def wait_store(j, p):
            pltpu.make_async_copy(
                t_b[p], out_hbm.at[pl.ds(2 * j, 2)].at[:, :, wid], ssem
            ).wait()

        crv = [jnp.right_shift(hh * 16 + iota, 3) for hh in range(DH)]
        civ = [jnp.bitwise_and(hh * 16 + iota, 7) for hh in range(DH)]

        def compute(j, p, tp):
            for h in range(2):
                l = 2 * j + h
                pev = [pe_v[l, pl.ds(hh * 16, 16)] for hh in range(DH)]
                th = t_b[tp].at[h]

                def rows4(r4, _):
                    for rr in range(4):
                        r = r4 * 4 + rr
                        bs = jnp.full((16,), r, jnp.int32)
                        for hh in range(DH):
                            v = g_b[p][h * 128 + r, pl.ds(hh * 16, 16)]
                            plsc.store_scatter(
                                th, [crv[hh], civ[hh], bs], v + pev[hh]
                            )
                    return 0

                lax.fori_loop(0, 32, rows4, 0)

        nj = seq // 2
        fire(0, 0)

        def step2(i, _):
            j0 = i * 2
            for q in range(2):
                j = j0 + q
                tp = q

                @pl.when(j + 1 < nj)
                def _():
                    fire(j + 1, 1 - q)

                @pl.when(j >= 2)
                def _():
                    wait_store(j - 2, tp)

                drain(j, q)
                compute(j, q, tp)
                store(j, tp)
            return 0

        lax.fori_loop(0, nj // 2, step2, 0)
        wait_store(nj - 2, 0)
        wait_store(nj - 1, 1)

    return k(table, x, pe)


def kernel(x, table):
    batch, seq = x.shape
    _, d = table.shape
    pe = jnp.asarray(_pe_table(_MAX_LEN, d)[:seq])
    out5 = _embed_pe(table, x, pe, batch=batch, seq=seq, d=d)
    return jnp.transpose(out5, (2, 4, 0, 1, 3)).reshape(batch, seq, d)


# speedup vs baseline: 1.2011x; 1.0007x over previous
import functools
import math

import jax
import jax.numpy as jnp
import numpy as np
from jax import lax
from jax.experimental import pallas as pl
from jax.experimental.pallas import tpu as pltpu
from jax.experimental.pallas import tpu_sc as plsc

_MAX_LEN = 200


def _pe_table(max_len, d_model):
    position = np.arange(0, max_len, dtype=np.float32)[:, None]
    div_term = np.exp(
        np.arange(0, d_model, 2, dtype=np.float32) * (-math.log(10000.0) / d_model)
    )
    pe = np.zeros((max_len, d_model), dtype=np.float32)
    pe[:, 0::2] = np.sin(position * div_term)
    if d_model % 2 == 1:
        pe[:, 1::2] = np.cos(position * div_term[:-1])
    else:
        pe[:, 1::2] = np.cos(position * div_term)
    return pe


@functools.partial(jax.jit, static_argnames=("batch", "seq", "d"))
def _embed_pe(table, x, pe, *, batch, seq, d):
    NC, NS = 2, 16
    NW = NC * NS
    assert batch == NW * 128, batch
    assert d % 16 == 0, d
    CR = d // 8
    DH = d // 16
    assert seq % 8 == 0, seq

    mesh = plsc.VectorSubcoreMesh(core_axis_name="c", subcore_axis_name="s")

    @functools.partial(
        pl.kernel,
        mesh=mesh,
        out_type=jax.ShapeDtypeStruct((seq, CR, NW, 8, 128), jnp.float32),
        compiler_params=pltpu.CompilerParams(
            use_tc_tiling_on_sc=False, needs_layout_passes=False
        ),
        scratch_types=[
            pltpu.VMEM((128, seq), jnp.int32),
            pltpu.VMEM((seq * 128,), jnp.int32),
            pltpu.VMEM((256, d), jnp.float32),
            pltpu.VMEM((256, d), jnp.float32),
            pltpu.VMEM((2, CR, 8, 128), jnp.float32),
            pltpu.VMEM((2, CR, 8, 128), jnp.float32),
            pltpu.VMEM((seq, d), jnp.float32),
            pltpu.SemaphoreType.DMA,
            pltpu.SemaphoreType.DMA,
        ],
    )
    def k(table_hbm, x_hbm, pe_hbm, out_hbm,
          xin, idxT, g0, g1, t0, t1, pe_v, gsem, ssem):
        g_b = (g0, g1)
        t_b = (t0, t1)
        wid = lax.axis_index("s") * NC + lax.axis_index("c")
        wb = wid * 128

        pltpu.sync_copy(pe_hbm, pe_v)
        pltpu.sync_copy(x_hbm.at[pl.ds(wb, 128)], xin)

        iota = lax.iota(jnp.int32, 16)

        def tr_body(l, _):
            cols = jnp.full((16,), l, jnp.int32)
            for bb in range(8):
                v = plsc.load_gather(xin, [bb * 16 + iota, cols])
                idxT[pl.ds(l * 128 + bb * 16, 16)] = v
            return 0

        lax.fori_loop(0, seq, tr_body, 0)

        def fire(j, p):
            pltpu.async_copy(
                table_hbm.at[idxT.at[pl.ds(j * 256, 256)]], g_b[p], gsem
            )

        def drain(j, p):
            pltpu.make_async_copy(
                table_hbm.at[idxT.at[pl.ds(j * 256, 256)]], g_b[p], gsem
            ).wait()

        def store(j, p):
            pltpu.async_copy(
                t_b[p], out_hbm.at[pl.ds(2 * j, 2)].at[:, :, wid], ssem
            )

        def wait_store(j, p):
            pltpu.make_async_copy(
                t_b[p], out_hbm.at[pl.ds(2 * j, 2)].at[:, :, wid], ssem
            ).wait()

        crv = [jnp.right_shift(hh * 16 + iota, 3) for hh in range(DH)]
        civ = [jnp.bitwise_and(hh * 16 + iota, 7) for hh in range(DH)]

        def compute(j, p, tp):
            for h in range(2):
                l = 2 * j + h
                pev = [pe_v[l, pl.ds(hh * 16, 16)] for hh in range(DH)]
                th = t_b[tp].at[h]

                def rows4(r4, _):
                    for rr in range(4):
                        r = r4 * 4 + rr
                        bs = jnp.full((16,), r, jnp.int32)
                        for hh in range(DH):
                            v = g_b[p][h * 128 + r, pl.ds(hh * 16, 16)]
                            plsc.store_scatter(
                                th, [crv[hh], civ[hh], bs], v + pev[hh]
                            )
                    return 0

                lax.fori_loop(0, 32, rows4, 0)

        nj = seq // 2
        fire(0, 0)

        def step2(i, _):
            j0 = i * 2
            for q in range(2):
                j = j0 + q
                tp = q

                @pl.when(j + 1 < nj)
                def _():
                    fire(j + 1, 1 - q)

                @pl.when(j >= 2)
                def _():
                    wait_store(j - 2, tp)

                drain(j, q)
                compute(j, q, tp)
                store(j, tp)
            return 0

        lax.fori_loop(0, nj // 2, step2, 0)
        wait_store(nj - 2, 0)
        wait_store(nj - 1, 1)

    return k(table, x, pe)


def kernel(x, table):
    batch, seq = x.shape
    _, d = table.shape
    pe = jnp.asarray(_pe_table(_MAX_LEN, d)[:seq])
    out5 = _embed_pe(table, x, pe, batch=batch, seq=seq, d=d)
    return jnp.transpose(out5, (2, 4, 0, 1, 3)).reshape(batch, seq, d)
